# baseline (speedup 1.0000x reference)
import functools

import jax
import jax.numpy as jnp
from jax import lax
from jax.experimental import pallas as pl
from jax.experimental.pallas import tpu as pltpu
from jax.experimental.pallas import tpu_sc as plsc

_N = 10000
_E = 160000
_TILE = 640

_INTERPRET = False

_SC_INFO = plsc.get_sparse_core_info()
_NC = _SC_INFO.num_cores
_NW = _SC_INFO.num_cores * _SC_INFO.num_subcores


def _sc_gather(table, idx, chunk):
    b = idx.shape[0]
    d = table.shape[1]
    bpw = b // _NW
    nch = bpw // chunk
    mesh = plsc.VectorSubcoreMesh(core_axis_name="c", subcore_axis_name="s")

    @functools.partial(
        pl.kernel,
        mesh=mesh,
        out_type=jax.ShapeDtypeStruct((b, d), table.dtype),
        scratch_types=[
            pltpu.VMEM((chunk,), jnp.int32),
            pltpu.VMEM((chunk, d), table.dtype),
            pltpu.SemaphoreType.DMA,
        ],
        compiler_params=pltpu.CompilerParams(use_tc_tiling_on_sc=False),
    )
    def gk(table_hbm, idx_hbm, out_hbm, idx_v, rows_v, sem):
        wid = lax.axis_index("s") * _NC + lax.axis_index("c")
        base = wid * bpw
        for i in range(nch):
            off = base + i * chunk
            pltpu.sync_copy(idx_hbm.at[pl.ds(off, chunk)], idx_v)
            pltpu.async_copy(table_hbm.at[idx_v], rows_v, sem).wait()
            pltpu.sync_copy(rows_v, out_hbm.at[pl.ds(off, chunk)])

    return gk(table, idx)


def _sc_scatter_add(vals, idx, v_pad, chunk):
    e = idx.shape[0]
    d = 16 if vals is None else vals.shape[1]
    ns = _SC_INFO.num_subcores
    epc = e // _NC
    epw = epc // ns
    nch = epw // chunk
    rows_out = v_pad // ns
    mesh = plsc.VectorSubcoreMesh(core_axis_name="c", subcore_axis_name="s")
    const_ones = vals is None
    if const_ones:
        vals = jnp.ones((chunk, d), jnp.float32)
    zeros = jnp.zeros((v_pad, d), jnp.float32)

    @functools.partial(
        pl.kernel,
        mesh=mesh,
        out_type=jax.ShapeDtypeStruct((_NC, v_pad, d), jnp.float32),
        scratch_types=[
            pltpu.VMEM((chunk,), jnp.int32),
            pltpu.VMEM((chunk, d), jnp.float32),
            pltpu.VMEM_SHARED((v_pad, d), jnp.float32),
            pltpu.SemaphoreType.DMA,
        ],
        compiler_params=pltpu.CompilerParams(use_tc_tiling_on_sc=False),
    )
    def sk(vals_hbm, idx_hbm, zeros_hbm, out_hbm, idx_v, rows_v, table_sh, sem):
        cid = lax.axis_index("c")
        sid = lax.axis_index("s")
        pltpu.sync_copy(zeros_hbm.at[pl.ds(sid * rows_out, rows_out)],
                        table_sh.at[pl.ds(sid * rows_out, rows_out)])
        if const_ones:
            pltpu.sync_copy(vals_hbm, rows_v)
        plsc.subcore_barrier()
        base = cid * epc + sid * epw
        for i in range(nch):
            off = base + i * chunk
            pltpu.sync_copy(idx_hbm.at[pl.ds(off, chunk)], idx_v)
            if not const_ones:
                pltpu.sync_copy(vals_hbm.at[pl.ds(off, chunk)], rows_v)
            pltpu.sync_copy(rows_v, table_sh.at[idx_v], add=True)
        plsc.subcore_barrier()
        pltpu.sync_copy(table_sh.at[pl.ds(sid * rows_out, rows_out)],
                        out_hbm.at[cid, pl.ds(sid * rows_out, rows_out)])

    return sk(vals, idx, zeros)


def _sc_gather_scalar(table1d, idx):
    b = idx.shape[0]
    v = table1d.shape[0]
    bpw = b // _NW
    mesh = plsc.VectorSubcoreMesh(core_axis_name="c", subcore_axis_name="s")

    @functools.partial(
        pl.kernel,
        mesh=mesh,
        out_type=jax.ShapeDtypeStruct((b,), table1d.dtype),
        scratch_types=[
            pltpu.VMEM((v,), table1d.dtype),
            pltpu.VMEM((bpw,), jnp.int32),
            pltpu.VMEM((bpw,), table1d.dtype),
        ],
        compiler_params=pltpu.CompilerParams(
            use_tc_tiling_on_sc=False, needs_layout_passes=False),
    )
    def gk(tab_hbm, idx_hbm, out_hbm, tab_v, idx_v, out_v):
        wid = lax.axis_index("s") * _NC + lax.axis_index("c")
        base = wid * bpw
        pltpu.sync_copy(tab_hbm, tab_v)
        pltpu.sync_copy(idx_hbm.at[pl.ds(base, bpw)], idx_v)

        def body(i, carry):
            ii = idx_v[pl.ds(i * 16, 16)]
            out_v[pl.ds(i * 16, 16)] = plsc.load_gather(tab_v, [ii])
            return carry

        lax.fori_loop(0, bpw // 16, body, 0)
        pltpu.sync_copy(out_v, out_hbm.at[pl.ds(base, bpw)])

    return gk(table1d, idx)


def _edge1_body(ea, xs, ps, pd, w1t, b1, w2t, b2, msg, ew):
    h = jnp.maximum(
        jnp.dot(ea[...], w1t[...], preferred_element_type=jnp.float32) + b1[...], 0.0)
    we = jnp.dot(h, w2t[...], preferred_element_type=jnp.float32) + b2[...]
    m = we * xs[...]
    msg[...] = jnp.concatenate(
        [m, jnp.ones((m.shape[0], 1), jnp.float32),
         jnp.zeros((m.shape[0], 15), jnp.float32)], axis=1)
    dp = ps[...] - pd[...]
    ew[...] = jnp.sqrt(jnp.sum(dp * dp, axis=1, keepdims=True))


def _conv1_edges(ea, xs, ps, pd, w1t, b1, w2t, b2):
    grid = _E // _TILE
    return pl.pallas_call(
        _edge1_body,
        grid=(grid,),
        in_specs=[
            pl.BlockSpec((_TILE, 2), lambda i: (i, 0)),
            pl.BlockSpec((_TILE, 1), lambda i: (i, 0)),
            pl.BlockSpec((_TILE, 2), lambda i: (i, 0)),
            pl.BlockSpec((_TILE, 2), lambda i: (i, 0)),
            pl.BlockSpec((2, 25), lambda i: (0, 0)),
            pl.BlockSpec((1, 25), lambda i: (0, 0)),
            pl.BlockSpec((25, 32), lambda i: (0, 0)),
            pl.BlockSpec((1, 32), lambda i: (0, 0)),
        ],
        out_specs=[
            pl.BlockSpec((_TILE, 48), lambda i: (i, 0)),
            pl.BlockSpec((_TILE, 1), lambda i: (i, 0)),
        ],
        out_shape=[
            jax.ShapeDtypeStruct((_E, 48), jnp.float32),
            jax.ShapeDtypeStruct((_E, 1), jnp.float32),
        ],
        interpret=_INTERPRET,
    )(ea, xs, ps, pd, w1t, b1, w2t, b2)


def _edge2_body(ea, xg, ps, pd, w1t, b1, w2f, brs, msg, msgb, ew):
    h = jnp.maximum(
        jnp.dot(ea[...], w1t[...], preferred_element_type=jnp.float32) + b1[...], 0.0)
    xgv = xg[...]
    we = jnp.dot(h, w2f[...], preferred_element_type=jnp.float32)
    acc = jnp.dot(xgv, brs[...], preferred_element_type=jnp.float32)
    for i in range(32):
        acc = acc + xgv[:, i:i + 1] * we[:, i * 64:(i + 1) * 64]
    msg[...] = acc[:, 0:48]
    msgb[...] = jnp.concatenate(
        [acc[:, 48:64], jnp.ones((acc.shape[0], 1), jnp.float32),
         jnp.zeros((acc.shape[0], 15), jnp.float32)], axis=1)
    dp = ps[...] - pd[...]
    ew[...] = jnp.sqrt(jnp.sum(dp * dp, axis=1, keepdims=True))


def _conv2_edges(ea, xg, ps, pd, w1t, b1, w2r, brs):
    grid = _E // _TILE
    return pl.pallas_call(
        _edge2_body,
        grid=(grid,),
        in_specs=[
            pl.BlockSpec((_TILE, 2), lambda i: (i, 0)),
            pl.BlockSpec((_TILE, 32), lambda i: (i, 0)),
            pl.BlockSpec((_TILE, 2), lambda i: (i, 0)),
            pl.BlockSpec((_TILE, 2), lambda i: (i, 0)),
            pl.BlockSpec((2, 25), lambda i: (0, 0)),
            pl.BlockSpec((1, 25), lambda i: (0, 0)),
            pl.BlockSpec((25, 2048), lambda i: (0, 0)),
            pl.BlockSpec((32, 64), lambda i: (0, 0)),
        ],
        out_specs=[
            pl.BlockSpec((_TILE, 48), lambda i: (i, 0)),
            pl.BlockSpec((_TILE, 32), lambda i: (i, 0)),
            pl.BlockSpec((_TILE, 1), lambda i: (i, 0)),
        ],
        out_shape=[
            jax.ShapeDtypeStruct((_E, 48), jnp.float32),
            jax.ShapeDtypeStruct((_E, 32), jnp.float32),
            jax.ShapeDtypeStruct((_E, 1), jnp.float32),
        ],
        interpret=_INTERPRET,
    )(ea, xg, ps, pd, w1t, b1, w2r, brs)


def _rank_relabel(raw, length):
    iota = jnp.arange(length, dtype=jnp.int32)
    sraw, order = jax.lax.sort((raw, iota), num_keys=1)
    first = jnp.concatenate(
        [jnp.ones((1,), jnp.bool_), sraw[1:] != sraw[:-1]])
    rank = (jnp.cumsum(first.astype(jnp.int32)) - 1).astype(jnp.int32)
    inv = jnp.zeros((length,), jnp.int32).at[order].set(rank)
    count = jnp.sum(first.astype(jnp.int32))
    return inv, count


def _graclus(src, dst, w, best_src, n):
    is_best = w >= best_src - 1e-12
    cand = jnp.where(is_best, dst, -1)
    partner = jax.ops.segment_max(cand, src, num_segments=n)
    idx = jnp.arange(n, dtype=jnp.int32)
    pp = jnp.where(partner >= 0, partner, idx).astype(jnp.int32)
    mutual = (pp[pp] == idx) & (pp != idx)
    raw = jnp.where(mutual, jnp.minimum(idx, pp), idx)
    return _rank_relabel(raw, n)


def kernel(x, pos, edge_index, edge_attr, nn1_w1, nn1_b1, nn1_w2, nn1_b2,
           conv1_root, conv1_bias, nn2_w1, nn2_b1, nn2_w2, nn2_b2,
           conv2_root, conv2_bias, fc1_w, fc1_b, fc2_w, fc2_b):
    n = _N
    src, dst = edge_index[0], edge_index[1]
    cat_sd = jnp.concatenate([src, dst])

    t1 = jnp.concatenate(
        [x, pos, jnp.zeros((n, 13), jnp.float32)], axis=1)
    g1 = _sc_gather(t1, cat_sd, 2000)
    xs = g1[:_E, 0:1]
    ps = g1[:_E, 1:3]
    pd = g1[_E:, 1:3]

    msg1, ew = _conv1_edges(
        edge_attr, xs, ps, pd,
        nn1_w1.T, nn1_b1.reshape(1, 25), nn1_w2.T, nn1_b2.reshape(1, 32))
    ew = ew[:, 0]
    part1 = _sc_scatter_add(msg1, dst, 10016, 1000)
    tab1 = part1[0] + part1[1]
    aggr1 = tab1[:n, 0:32] / jnp.maximum(tab1[:n, 32:33], 1.0)
    x1 = jax.nn.elu(x @ conv1_root + aggr1 + conv1_bias)

    dpart1 = _sc_scatter_add(None, src, 10016, 1000)
    deg1 = (dpart1[0] + dpart1[1])[:n, 0]
    inv1 = 1.0 / jnp.maximum(deg1, 1.0)
    invg = _sc_gather_scalar(inv1, cat_sd)
    w = ew * (invg[:_E] + invg[_E:])
    best1 = jax.ops.segment_max(w, src, num_segments=n)
    bestg = _sc_gather_scalar(best1, src)
    cluster1, c1 = _graclus(src, dst, w, bestg, n)

    xp = jax.ops.segment_max(x1, cluster1, num_segments=n + 1)
    ncnt = jax.ops.segment_sum(jnp.ones((n,), jnp.float32), cluster1,
                               num_segments=n + 1)
    posp = jax.ops.segment_sum(pos, cluster1, num_segments=n + 1) \
        / jnp.maximum(ncnt, 1.0)[:, None]

    cg = _sc_gather_scalar(cluster1, cat_sd)
    s2c = cg[:_E]
    d2c = cg[_E:]
    keys = jnp.where(s2c != d2c, s2c * n + d2c, -1)
    sk = jnp.sort(keys)
    firstocc = jnp.concatenate(
        [jnp.ones((1,), jnp.bool_), sk[1:] != sk[:-1]]) & (sk >= 0)
    s2 = jnp.where(firstocc, sk // n, n).astype(jnp.int32)
    d2 = jnp.where(firstocc, sk % n, n).astype(jnp.int32)
    cat_sd2 = jnp.concatenate([s2, d2])

    pospt = jnp.concatenate(
        [posp, jnp.zeros((n + 1, 14), jnp.float32)], axis=1)
    g4 = _sc_gather(pospt, cat_sd2, 2000)
    psrc2 = g4[:_E, 0:2]
    pdst2 = g4[_E:, 0:2]
    cart = psrc2 - pdst2
    mx = jnp.max(jnp.abs(cart))
    ea2 = cart / (2.0 * mx) + 0.5
    xg = _sc_gather(xp, cat_sd2[:_E], 1000)

    w2f = nn2_w2.T
    brs = nn2_b2.reshape(32, 64)
    msg2, msg2b, ew2 = _conv2_edges(
        ea2, xg, psrc2, pdst2,
        nn2_w1.T, nn2_b1.reshape(1, 25), w2f, brs)
    ew2 = ew2[:, 0]
    part2 = _sc_scatter_add(msg2, d2, 10016, 1000)
    part2b = _sc_scatter_add(msg2b, d2, 10016, 1000)
    tab2 = part2[0] + part2[1]
    tab2b = part2b[0] + part2b[1]
    s2full = jnp.concatenate([tab2[:n + 1, 0:48], tab2b[:n + 1, 0:16]], axis=1)
    aggr2 = s2full / jnp.maximum(tab2b[:n + 1, 16:17], 1.0)
    x2 = jax.nn.elu(xp @ conv2_root + aggr2 + conv2_bias)

    dpart2 = _sc_scatter_add(None, s2, 10016, 1000)
    deg2 = (dpart2[0] + dpart2[1])[:n + 1, 0]
    inv2 = 1.0 / jnp.maximum(deg2, 1.0)
    invg2 = _sc_gather_scalar(inv2, cat_sd2)
    w2 = ew2 * (invg2[:_E] + invg2[_E:])
    best2 = jax.ops.segment_max(w2, s2, num_segments=n + 1)
    bestg2 = _sc_gather_scalar(best2, s2)
    cluster2, _ = _graclus(s2, d2, w2, bestg2, n + 1)
    x3 = jax.ops.segment_max(x2, cluster2, num_segments=n + 1)

    c2 = jnp.max(jnp.where(jnp.arange(n + 1) < c1, cluster2, -1)) + 1
    row_valid = jnp.arange(n + 1) < c2
    g = jnp.sum(jnp.where(row_valid[:, None], x3, 0.0), axis=0,
                keepdims=True) / c2.astype(jnp.float32)
    h = jax.nn.elu(g @ fc1_w.T + fc1_b)
    return jax.nn.log_softmax(h @ fc2_w.T + fc2_b, axis=1)


# speedup vs baseline: 1.1340x; 1.1340x over previous
import functools

import jax
import jax.numpy as jnp
from jax import lax
from jax.experimental import pallas as pl
from jax.experimental.pallas import tpu as pltpu
from jax.experimental.pallas import tpu_sc as plsc

_N = 10000
_E = 160000
_TILE = 640

_SC_INFO = plsc.get_sparse_core_info()
_NC = _SC_INFO.num_cores
_NW = _SC_INFO.num_cores * _SC_INFO.num_subcores


def _sc_gather(table, idx, chunk):
    b = idx.shape[0]
    d = table.shape[1]
    bpw = b // _NW
    nch = bpw // chunk
    mesh = plsc.VectorSubcoreMesh(core_axis_name="c", subcore_axis_name="s")

    @functools.partial(
        pl.kernel,
        mesh=mesh,
        out_type=jax.ShapeDtypeStruct((b, d), table.dtype),
        scratch_types=[
            pltpu.VMEM((chunk,), jnp.int32),
            pltpu.VMEM((chunk, d), table.dtype),
            pltpu.SemaphoreType.DMA,
        ],
        compiler_params=pltpu.CompilerParams(use_tc_tiling_on_sc=False),
    )
    def gk(table_hbm, idx_hbm, out_hbm, idx_v, rows_v, sem):
        wid = lax.axis_index("s") * _NC + lax.axis_index("c")
        base = wid * bpw
        for i in range(nch):
            off = base + i * chunk
            pltpu.sync_copy(idx_hbm.at[pl.ds(off, chunk)], idx_v)
            pltpu.async_copy(table_hbm.at[idx_v], rows_v, sem).wait()
            pltpu.sync_copy(rows_v, out_hbm.at[pl.ds(off, chunk)])

    return gk(table, idx)


def _sc_scatter_add(vals, idx, v_pad, chunk):
    e = idx.shape[0]
    d = 16 if vals is None else vals.shape[1]
    ns = _SC_INFO.num_subcores
    epc = e // _NC
    epw = epc // ns
    nch = epw // chunk
    rows_out = v_pad // ns
    mesh = plsc.VectorSubcoreMesh(core_axis_name="c", subcore_axis_name="s")
    const_ones = vals is None
    if const_ones:
        vals = jnp.ones((chunk, d), jnp.float32)
    zeros = jnp.zeros((v_pad, d), jnp.float32)

    @functools.partial(
        pl.kernel,
        mesh=mesh,
        out_type=jax.ShapeDtypeStruct((_NC, v_pad, d), jnp.float32),
        scratch_types=[
            pltpu.VMEM((chunk,), jnp.int32),
            pltpu.VMEM((chunk, d), jnp.float32),
            pltpu.VMEM_SHARED((v_pad, d), jnp.float32),
            pltpu.SemaphoreType.DMA,
        ],
        compiler_params=pltpu.CompilerParams(use_tc_tiling_on_sc=False),
    )
    def sk(vals_hbm, idx_hbm, zeros_hbm, out_hbm, idx_v, rows_v, table_sh, sem):
        cid = lax.axis_index("c")
        sid = lax.axis_index("s")
        pltpu.sync_copy(zeros_hbm.at[pl.ds(sid * rows_out, rows_out)],
                        table_sh.at[pl.ds(sid * rows_out, rows_out)])
        if const_ones:
            pltpu.sync_copy(vals_hbm, rows_v)
        plsc.subcore_barrier()
        base = cid * epc + sid * epw
        for i in range(nch):
            off = base + i * chunk
            pltpu.sync_copy(idx_hbm.at[pl.ds(off, chunk)], idx_v)
            if not const_ones:
                pltpu.sync_copy(vals_hbm.at[pl.ds(off, chunk)], rows_v)
            pltpu.sync_copy(rows_v, table_sh.at[idx_v], add=True)
        plsc.subcore_barrier()
        pltpu.sync_copy(table_sh.at[pl.ds(sid * rows_out, rows_out)],
                        out_hbm.at[cid, pl.ds(sid * rows_out, rows_out)])

    return sk(vals, idx, zeros)


def _sc_gather_scalar(table1d, idx):
    b = idx.shape[0]
    v = table1d.shape[0]
    bpw = b // _NW
    mesh = plsc.VectorSubcoreMesh(core_axis_name="c", subcore_axis_name="s")

    @functools.partial(
        pl.kernel,
        mesh=mesh,
        out_type=jax.ShapeDtypeStruct((b,), table1d.dtype),
        scratch_types=[
            pltpu.VMEM((v,), table1d.dtype),
            pltpu.VMEM((bpw,), jnp.int32),
            pltpu.VMEM((bpw,), table1d.dtype),
        ],
        compiler_params=pltpu.CompilerParams(
            use_tc_tiling_on_sc=False, needs_layout_passes=False),
    )
    def gk(tab_hbm, idx_hbm, out_hbm, tab_v, idx_v, out_v):
        wid = lax.axis_index("s") * _NC + lax.axis_index("c")
        base = wid * bpw
        pltpu.sync_copy(tab_hbm, tab_v)
        pltpu.sync_copy(idx_hbm.at[pl.ds(base, bpw)], idx_v)

        def body(i, carry):
            ii = idx_v[pl.ds(i * 16, 16)]
            out_v[pl.ds(i * 16, 16)] = plsc.load_gather(tab_v, [ii])
            return carry

        lax.fori_loop(0, bpw // 16, body, 0)
        pltpu.sync_copy(out_v, out_hbm.at[pl.ds(base, bpw)])

    return gk(table1d, idx)


def _edge1_body(ea, xs, ps, pd, w1t, b1, w2t, b2, msg, ew):
    h = jnp.maximum(
        jnp.dot(ea[...], w1t[...], preferred_element_type=jnp.float32) + b1[...], 0.0)
    we = jnp.dot(h, w2t[...], preferred_element_type=jnp.float32) + b2[...]
    m = we * xs[...]
    msg[...] = jnp.concatenate(
        [m, jnp.ones((m.shape[0], 1), jnp.float32),
         jnp.zeros((m.shape[0], 15), jnp.float32)], axis=1)
    dp = ps[...] - pd[...]
    ew[...] = jnp.sqrt(jnp.sum(dp * dp, axis=1, keepdims=True))


def _conv1_edges(ea, xs, ps, pd, w1t, b1, w2t, b2):
    grid = _E // _TILE
    return pl.pallas_call(
        _edge1_body,
        grid=(grid,),
        in_specs=[
            pl.BlockSpec((_TILE, 2), lambda i: (i, 0)),
            pl.BlockSpec((_TILE, 1), lambda i: (i, 0)),
            pl.BlockSpec((_TILE, 2), lambda i: (i, 0)),
            pl.BlockSpec((_TILE, 2), lambda i: (i, 0)),
            pl.BlockSpec((2, 25), lambda i: (0, 0)),
            pl.BlockSpec((1, 25), lambda i: (0, 0)),
            pl.BlockSpec((25, 32), lambda i: (0, 0)),
            pl.BlockSpec((1, 32), lambda i: (0, 0)),
        ],
        out_specs=[
            pl.BlockSpec((_TILE, 48), lambda i: (i, 0)),
            pl.BlockSpec((_TILE, 1), lambda i: (i, 0)),
        ],
        out_shape=[
            jax.ShapeDtypeStruct((_E, 48), jnp.float32),
            jax.ShapeDtypeStruct((_E, 1), jnp.float32),
        ],
    )(ea, xs, ps, pd, w1t, b1, w2t, b2)


def _edge2_body(ea, xg, ps, pd, w1t, b1, w2r, brs, msg, msgb, ew):
    h = jnp.maximum(
        jnp.dot(ea[...], w1t[...], preferred_element_type=jnp.float32) + b1[...], 0.0)
    xgv = xg[...]
    acc = jnp.dot(xgv, brs[...], preferred_element_type=jnp.float32)
    for k in range(25):
        acc = acc + h[:, k:k + 1] * jnp.dot(
            xgv, w2r[k], preferred_element_type=jnp.float32)
    msg[...] = acc[:, 0:48]
    msgb[...] = jnp.concatenate(
        [acc[:, 48:64], jnp.ones((acc.shape[0], 1), jnp.float32),
         jnp.zeros((acc.shape[0], 15), jnp.float32)], axis=1)
    dp = ps[...] - pd[...]
    ew[...] = jnp.sqrt(jnp.sum(dp * dp, axis=1, keepdims=True))


def _conv2_edges(ea, xg, ps, pd, w1t, b1, w2r, brs):
    grid = _E // _TILE
    return pl.pallas_call(
        _edge2_body,
        grid=(grid,),
        in_specs=[
            pl.BlockSpec((_TILE, 2), lambda i: (i, 0)),
            pl.BlockSpec((_TILE, 32), lambda i: (i, 0)),
            pl.BlockSpec((_TILE, 2), lambda i: (i, 0)),
            pl.BlockSpec((_TILE, 2), lambda i: (i, 0)),
            pl.BlockSpec((2, 25), lambda i: (0, 0)),
            pl.BlockSpec((1, 25), lambda i: (0, 0)),
            pl.BlockSpec((25, 32, 64), lambda i: (0, 0, 0)),
            pl.BlockSpec((32, 64), lambda i: (0, 0)),
        ],
        out_specs=[
            pl.BlockSpec((_TILE, 48), lambda i: (i, 0)),
            pl.BlockSpec((_TILE, 32), lambda i: (i, 0)),
            pl.BlockSpec((_TILE, 1), lambda i: (i, 0)),
        ],
        out_shape=[
            jax.ShapeDtypeStruct((_E, 48), jnp.float32),
            jax.ShapeDtypeStruct((_E, 32), jnp.float32),
            jax.ShapeDtypeStruct((_E, 1), jnp.float32),
        ],
    )(ea, xg, ps, pd, w1t, b1, w2r, brs)


def _rank_relabel(raw, length):
    iota = jnp.arange(length, dtype=jnp.int32)
    sraw, order = jax.lax.sort((raw, iota), num_keys=1)
    first = jnp.concatenate(
        [jnp.ones((1,), jnp.bool_), sraw[1:] != sraw[:-1]])
    rank = (jnp.cumsum(first.astype(jnp.int32)) - 1).astype(jnp.int32)
    inv = jnp.zeros((length,), jnp.int32).at[order].set(rank)
    count = jnp.sum(first.astype(jnp.int32))
    return inv, count


def _graclus(src, dst, w, best_src, n):
    is_best = w >= best_src - 1e-12
    cand = jnp.where(is_best, dst, -1)
    partner = jax.ops.segment_max(cand, src, num_segments=n)
    idx = jnp.arange(n, dtype=jnp.int32)
    pp = jnp.where(partner >= 0, partner, idx).astype(jnp.int32)
    mutual = (pp[pp] == idx) & (pp != idx)
    raw = jnp.where(mutual, jnp.minimum(idx, pp), idx)
    return _rank_relabel(raw, n)


def kernel(x, pos, edge_index, edge_attr, nn1_w1, nn1_b1, nn1_w2, nn1_b2,
           conv1_root, conv1_bias, nn2_w1, nn2_b1, nn2_w2, nn2_b2,
           conv2_root, conv2_bias, fc1_w, fc1_b, fc2_w, fc2_b):
    n = _N
    src, dst = edge_index[0], edge_index[1]
    cat_sd = jnp.concatenate([src, dst])

    t1 = jnp.concatenate(
        [x, pos, jnp.zeros((n, 13), jnp.float32)], axis=1)
    g1 = _sc_gather(t1, cat_sd, 2000)
    xs = g1[:_E, 0:1]
    ps = g1[:_E, 1:3]
    pd = g1[_E:, 1:3]

    msg1, ew = _conv1_edges(
        edge_attr, xs, ps, pd,
        nn1_w1.T, nn1_b1.reshape(1, 25), nn1_w2.T, nn1_b2.reshape(1, 32))
    ew = ew[:, 0]
    part1 = _sc_scatter_add(msg1, dst, 10016, 1000)
    tab1 = part1[0] + part1[1]
    aggr1 = tab1[:n, 0:32] / jnp.maximum(tab1[:n, 32:33], 1.0)
    x1 = jax.nn.elu(x @ conv1_root + aggr1 + conv1_bias)

    dpart1 = _sc_scatter_add(None, src, 10016, 1000)
    deg1 = (dpart1[0] + dpart1[1])[:n, 0]
    inv1 = 1.0 / jnp.maximum(deg1, 1.0)
    invg = _sc_gather_scalar(inv1, cat_sd)
    w = ew * (invg[:_E] + invg[_E:])
    best1 = jax.ops.segment_max(w, src, num_segments=n)
    bestg = _sc_gather_scalar(best1, src)
    cluster1, c1 = _graclus(src, dst, w, bestg, n)

    xp = jax.ops.segment_max(x1, cluster1, num_segments=n + 1)
    ncnt = jax.ops.segment_sum(jnp.ones((n,), jnp.float32), cluster1,
                               num_segments=n + 1)
    posp = jax.ops.segment_sum(pos, cluster1, num_segments=n + 1) \
        / jnp.maximum(ncnt, 1.0)[:, None]

    cg = _sc_gather_scalar(cluster1, cat_sd)
    s2c = cg[:_E]
    d2c = cg[_E:]
    keys = jnp.where(s2c != d2c, s2c * n + d2c, -1)
    sk = jnp.sort(keys)
    firstocc = jnp.concatenate(
        [jnp.ones((1,), jnp.bool_), sk[1:] != sk[:-1]]) & (sk >= 0)
    s2 = jnp.where(firstocc, sk // n, n).astype(jnp.int32)
    d2 = jnp.where(firstocc, sk % n, n).astype(jnp.int32)
    cat_sd2 = jnp.concatenate([s2, d2])

    pospt = jnp.concatenate(
        [posp, jnp.zeros((n + 1, 14), jnp.float32)], axis=1)
    g4 = _sc_gather(pospt, cat_sd2, 2000)
    psrc2 = g4[:_E, 0:2]
    pdst2 = g4[_E:, 0:2]
    cart = psrc2 - pdst2
    mx = jnp.max(jnp.abs(cart))
    ea2 = cart / (2.0 * mx) + 0.5
    xg = _sc_gather(xp, cat_sd2[:_E], 1000)

    w2r = nn2_w2.T.reshape(25, 32, 64)
    brs = nn2_b2.reshape(32, 64)
    msg2, msg2b, ew2 = _conv2_edges(
        ea2, xg, psrc2, pdst2,
        nn2_w1.T, nn2_b1.reshape(1, 25), w2r, brs)
    ew2 = ew2[:, 0]
    part2 = _sc_scatter_add(msg2, d2, 10016, 1000)
    part2b = _sc_scatter_add(msg2b, d2, 10016, 1000)
    tab2 = part2[0] + part2[1]
    tab2b = part2b[0] + part2b[1]
    s2full = jnp.concatenate([tab2[:n + 1, 0:48], tab2b[:n + 1, 0:16]], axis=1)
    aggr2 = s2full / jnp.maximum(tab2b[:n + 1, 16:17], 1.0)
    x2 = jax.nn.elu(xp @ conv2_root + aggr2 + conv2_bias)

    dpart2 = _sc_scatter_add(None, s2, 10016, 1000)
    deg2 = (dpart2[0] + dpart2[1])[:n + 1, 0]
    inv2 = 1.0 / jnp.maximum(deg2, 1.0)
    invg2 = _sc_gather_scalar(inv2, cat_sd2)
    w2 = ew2 * (invg2[:_E] + invg2[_E:])
    best2 = jax.ops.segment_max(w2, s2, num_segments=n + 1)
    bestg2 = _sc_gather_scalar(best2, s2)
    cluster2, _ = _graclus(s2, d2, w2, bestg2, n + 1)
    x3 = jax.ops.segment_max(x2, cluster2, num_segments=n + 1)

    c2 = jnp.max(jnp.where(jnp.arange(n + 1) < c1, cluster2, -1)) + 1
    row_valid = jnp.arange(n + 1) < c2
    g = jnp.sum(jnp.where(row_valid[:, None], x3, 0.0), axis=0,
                keepdims=True) / c2.astype(jnp.float32)
    h = jax.nn.elu(g @ fc1_w.T + fc1_b)
    return jax.nn.log_softmax(h @ fc2_w.T + fc2_b, axis=1)
